# Initial kernel scaffold; baseline (speedup 1.0000x reference)
import jax
import jax.numpy as jnp
from jax.experimental import pallas as pl


def kernel(x, edge_index, W1, b1, W2, b2, W_ih, W_hh, b_ih, b_hh, fc_W, fc_b):
    raise NotImplementedError("write your pallas kernel here")


# speedup vs baseline: 11.9829x; 11.9829x over previous
import functools

import jax
import jax.numpy as jnp
from jax import lax
from jax.experimental import pallas as pl
from jax.experimental.pallas import tpu as pltpu
from jax.experimental.pallas import tpu_sc as plsc

NC = 2
NS = 16
NW = NC * NS
LANES = 16
K = 128
BLK = 512


def _build_sc_deg(Np, C):
  stripe = Np // NS

  @functools.partial(
      pl.kernel,
      out_type=jax.ShapeDtypeStruct((NC, Np), jnp.float32),
      mesh=plsc.VectorSubcoreMesh(core_axis_name="c", subcore_axis_name="s"),
      scratch_types=[
          pltpu.VMEM((C, K), jnp.int32),
          pltpu.VMEM((K,), jnp.float32),
          pltpu.VMEM((stripe,), jnp.float32),
          pltpu.VMEM_SHARED((Np,), jnp.float32),
      ],
  )
  def deg_kernel(dst_hbm, out_hbm, dst_v, ones_v, zero_v, acc_sh):
    c = lax.axis_index("c")
    s = lax.axis_index("s")
    wid = s * NC + c
    for i in range(K // LANES):
      ones_v[pl.ds(i * LANES, LANES)] = jnp.ones((LANES,), jnp.float32)
    for i in range(stripe // LANES):
      zero_v[pl.ds(i * LANES, LANES)] = jnp.zeros((LANES,), jnp.float32)
    pltpu.sync_copy(zero_v, acc_sh.at[pl.ds(s * stripe, stripe)])
    pltpu.sync_copy(dst_hbm.at[wid], dst_v)
    plsc.subcore_barrier()

    def body(j, carry):
      pltpu.sync_copy(ones_v, acc_sh.at[dst_v.at[j]], add=True)
      return carry

    lax.fori_loop(0, C, body, 0)
    plsc.subcore_barrier()
    pltpu.sync_copy(acc_sh.at[pl.ds(s * stripe, stripe)],
                    out_hbm.at[c, pl.ds(s * stripe, stripe)])

  return deg_kernel


def _build_sc_msg(Np, H, C):
  stripe = Np // NS

  @functools.partial(
      pl.kernel,
      out_type=jax.ShapeDtypeStruct((NC, Np, H), jnp.float32),
      mesh=plsc.VectorSubcoreMesh(core_axis_name="c", subcore_axis_name="s"),
      scratch_types=[
          pltpu.VMEM((C, K), jnp.int32),
          pltpu.VMEM((C, K), jnp.int32),
          pltpu.VMEM((K, H), jnp.float32),
          pltpu.VMEM_SHARED((Np, H), jnp.float32),
          pltpu.SemaphoreType.DMA,
      ],
  )
  def msg_kernel(y_hbm, src_hbm, dst_hbm, out_hbm, src_v, dst_v, rows_v,
                 acc_sh, sem):
    c = lax.axis_index("c")
    s = lax.axis_index("s")
    wid = s * NC + c
    pltpu.sync_copy(y_hbm.at[pl.ds(s * stripe, stripe)],
                    acc_sh.at[pl.ds(s * stripe, stripe)])
    pltpu.sync_copy(src_hbm.at[wid], src_v)
    pltpu.sync_copy(dst_hbm.at[wid], dst_v)
    plsc.subcore_barrier()

    def body(j, carry):
      pltpu.async_copy(y_hbm.at[src_v.at[j]], rows_v, sem).wait()
      pltpu.sync_copy(rows_v, acc_sh.at[dst_v.at[j]], add=True)
      return carry

    lax.fori_loop(0, C, body, 0)
    plsc.subcore_barrier()
    pltpu.sync_copy(acc_sh.at[pl.ds(s * stripe, stripe)],
                    out_hbm.at[c, pl.ds(s * stripe, stripe)])

  return msg_kernel


def _tc1(d0, d1, x, W1, Np, D, H):
  def body(d0_ref, d1_ref, x_ref, w_ref, dinv_ref, y_ref):
    deg = d0_ref[...] + d1_ref[...] + 1.0
    dinv = lax.rsqrt(deg)
    dinv_ref[...] = dinv
    y_ref[...] = jnp.dot(x_ref[...], w_ref[...],
                         preferred_element_type=jnp.float32) * dinv

  return pl.pallas_call(
      body,
      grid=(Np // BLK,),
      in_specs=[
          pl.BlockSpec((BLK, 1), lambda i: (i, 0)),
          pl.BlockSpec((BLK, 1), lambda i: (i, 0)),
          pl.BlockSpec((BLK, D), lambda i: (i, 0)),
          pl.BlockSpec((D, H), lambda i: (0, 0)),
      ],
      out_specs=[
          pl.BlockSpec((BLK, 1), lambda i: (i, 0)),
          pl.BlockSpec((BLK, H), lambda i: (i, 0)),
      ],
      out_shape=[
          jax.ShapeDtypeStruct((Np, 1), jnp.float32),
          jax.ShapeDtypeStruct((Np, H), jnp.float32),
      ],
  )(d0, d1, x, W1)


def _tc2(dinv, p0, p1, y1, b1, W2, Np, H):
  def body(dinv_ref, p0_ref, p1_ref, y1_ref, b1_ref, w_ref, y2_ref):
    dinv = dinv_ref[...]
    h = jnp.maximum(
        dinv * (p0_ref[...] + p1_ref[...] - y1_ref[...]) + b1_ref[...], 0.0)
    y2_ref[...] = jnp.dot(h, w_ref[...],
                          preferred_element_type=jnp.float32) * dinv

  return pl.pallas_call(
      body,
      grid=(Np // BLK,),
      in_specs=[
          pl.BlockSpec((BLK, 1), lambda i: (i, 0)),
          pl.BlockSpec((BLK, H), lambda i: (i, 0)),
          pl.BlockSpec((BLK, H), lambda i: (i, 0)),
          pl.BlockSpec((BLK, H), lambda i: (i, 0)),
          pl.BlockSpec((1, H), lambda i: (0, 0)),
          pl.BlockSpec((H, H), lambda i: (0, 0)),
      ],
      out_specs=pl.BlockSpec((BLK, H), lambda i: (i, 0)),
      out_shape=jax.ShapeDtypeStruct((Np, H), jnp.float32),
  )(dinv, p0, p1, y1, b1, W2)


def _tc3(dinv, q0, q1, y2, b2, Wih_T, bias, fcW_T, fcb, Np, H, OUT):
  def body(dinv_ref, q0_ref, q1_ref, y2_ref, b2_ref, wih_ref, bias_ref,
           fcw_ref, fcb_ref, out_ref):
    dinv = dinv_ref[...]
    h = jnp.maximum(
        dinv * (q0_ref[...] + q1_ref[...] - y2_ref[...]) + b2_ref[...], 0.0)
    g = jnp.dot(h, wih_ref[...], preferred_element_type=jnp.float32) \
        + bias_ref[...]
    gi = g[:, :H]
    gg = g[:, 2 * H:3 * H]
    go = g[:, 3 * H:]
    cell = jax.nn.sigmoid(gi) * jnp.tanh(gg)
    hh = jax.nn.sigmoid(go) * jnp.tanh(cell)
    out_ref[...] = jnp.dot(hh, fcw_ref[...],
                           preferred_element_type=jnp.float32) + fcb_ref[...]

  return pl.pallas_call(
      body,
      grid=(Np // BLK,),
      in_specs=[
          pl.BlockSpec((BLK, 1), lambda i: (i, 0)),
          pl.BlockSpec((BLK, H), lambda i: (i, 0)),
          pl.BlockSpec((BLK, H), lambda i: (i, 0)),
          pl.BlockSpec((BLK, H), lambda i: (i, 0)),
          pl.BlockSpec((1, H), lambda i: (0, 0)),
          pl.BlockSpec((H, 4 * H), lambda i: (0, 0)),
          pl.BlockSpec((1, 4 * H), lambda i: (0, 0)),
          pl.BlockSpec((H, OUT), lambda i: (0, 0)),
          pl.BlockSpec((1, OUT), lambda i: (0, 0)),
      ],
      out_specs=pl.BlockSpec((BLK, OUT), lambda i: (i, 0)),
      out_shape=jax.ShapeDtypeStruct((Np, OUT), jnp.float32),
  )(dinv, q0, q1, y2, b2, Wih_T, bias, fcW_T, fcb)


def kernel(x, edge_index, W1, b1, W2, b2, W_ih, W_hh, b_ih, b_hh, fc_W, fc_b):
  N, D = x.shape
  H = W1.shape[1]
  OUT = fc_W.shape[0]
  E = edge_index.shape[1]

  Np = ((N + BLK - 1) // BLK) * BLK
  EW = NW * K
  C = (E + EW - 1) // EW
  Ep = C * EW

  x_p = jnp.zeros((Np, D), jnp.float32).at[:N].set(x)
  fill = jnp.full((Ep - E,), N, jnp.int32)
  src_r = jnp.concatenate([edge_index[0].astype(jnp.int32), fill]) \
      .reshape(NW, C, K)
  dst_r = jnp.concatenate([edge_index[1].astype(jnp.int32), fill]) \
      .reshape(NW, C, K)

  sc_deg = _build_sc_deg(Np, C)
  sc_msg = _build_sc_msg(Np, H, C)

  deg = sc_deg(dst_r)
  d0 = deg[0].reshape(Np, 1)
  d1 = deg[1].reshape(Np, 1)

  dinv, y1 = _tc1(d0, d1, x_p, W1, Np, D, H)

  p = sc_msg(y1, src_r, dst_r)
  y2 = _tc2(dinv, p[0], p[1], y1, b1.reshape(1, H), W2, Np, H)

  q = sc_msg(y2, src_r, dst_r)
  out = _tc3(dinv, q[0], q[1], y2, b2.reshape(1, H), W_ih.T,
             (b_ih + b_hh).reshape(1, 4 * H), fc_W.T, fc_b.reshape(1, OUT),
             Np, H, OUT)
  return out[:N]
